# baseline (speedup 1.0000x reference)
import functools

import jax
import jax.numpy as jnp
from jax import lax
from jax.experimental import pallas as pl
from jax.experimental.pallas import tpu as pltpu
from jax.experimental.pallas import tpu_sc as plsc

DICT_SIZE = 1000000
EMBED_DIM = 64
BATCH = 16384
HIST = 20
TOTAL = BATCH * HIST

NUM_CORES = 2
NUM_SUBCORES = 16
NW = NUM_CORES * NUM_SUBCORES

PER_W = TOTAL // NW
CHUNK = 128
NCHUNK = PER_W // CHUNK
NSLOT = 10
LOOKAHEAD = 5

NFULL = DICT_SIZE // 128
TAIL = DICT_SIZE - NFULL * 128
BLK_FLOOR = NFULL // NW
BLK_EXTRA = NFULL % NW
TSLOT = 2

_mesh = plsc.VectorSubcoreMesh(core_axis_name="c", subcore_axis_name="s")


@functools.partial(
    pl.kernel,
    out_type=jax.ShapeDtypeStruct((DICT_SIZE // 2, 128), jnp.float32),
    mesh=_mesh,
    scratch_types=[
        pltpu.VMEM((TSLOT, 64, 128), jnp.float32),
        pltpu.VMEM((TSLOT, 64, 128), jnp.float32),
        [pltpu.SemaphoreType.DMA] * TSLOT,
        [pltpu.SemaphoreType.DMA] * TSLOT,
    ],
    compiler_params=pltpu.CompilerParams(
        use_tc_tiling_on_sc=True, needs_layout_passes=False
    ),
)
def _table_transpose(wt_hbm, tail_hbm, out_hbm, src_v, dst_v, lsems, ssems):
    wid = lax.axis_index("s") * NUM_CORES + lax.axis_index("c")
    blk0 = wid * BLK_FLOOR + jnp.minimum(wid, BLK_EXTRA)
    nb = BLK_FLOOR + jnp.where(wid < BLK_EXTRA, 1, 0)

    def load(slot, blk):
        pltpu.async_copy(
            wt_hbm.at[:, pl.ds(blk * 128, 128)], src_v.at[slot], lsems[slot]
        )

    def wait_load(slot):
        pltpu.make_async_copy(
            wt_hbm.at[:, pl.ds(0, 128)], src_v.at[slot], lsems[slot]
        ).wait()

    def store(slot, row0, nrows):
        pltpu.async_copy(
            dst_v.at[slot, pl.ds(0, nrows)],
            out_hbm.at[pl.ds(row0, nrows)],
            ssems[slot],
        )

    def wait_store(slot, nrows):
        pltpu.make_async_copy(
            dst_v.at[slot, pl.ds(0, nrows)],
            out_hbm.at[pl.ds(0, nrows)],
            ssems[slot],
        ).wait()

    def shuffle(slot, ngroups):
        for j in range(ngroups):
            lane = jax.lax.iota(jnp.int32, 16) + (16 * j)
            rowv = lane // 2
            colv0 = (lane % 2) * 64
            for d in range(64):
                v = src_v[slot, d, pl.ds(16 * j, 16)]
                plsc.store_scatter(dst_v.at[slot], [rowv, colv0 + d], v)

    for s in range(TSLOT):
        load(s, blk0 + s)

    @pl.loop(0, BLK_FLOOR + 1, step=TSLOT)
    def _group(g):
        for s in range(TSLOT):
            i = g + s

            @pl.when(i < nb)
            def _():
                wait_load(s)

                @pl.when(i >= TSLOT)
                def _():
                    wait_store(s, 64)

                shuffle(s, 8)
                store(s, (blk0 + i) * 64, 64)

                @pl.when(i + TSLOT < nb)
                def _():
                    load(s, blk0 + i + TSLOT)

    for s in range(TSLOT):
        wait_store(s, 64)

    @pl.when(wid == 0)
    def _():
        pltpu.async_copy(tail_hbm, src_v.at[0], lsems[0])
        pltpu.make_async_copy(tail_hbm, src_v.at[0], lsems[0]).wait()
        shuffle(0, 4)
        store(0, NFULL * 64, TAIL // 2)
        wait_store(0, TAIL // 2)


@functools.partial(
    pl.kernel,
    out_type=jax.ShapeDtypeStruct((TOTAL, EMBED_DIM), jnp.float32),
    mesh=_mesh,
    scratch_types=[
        pltpu.VMEM((NCHUNK, CHUNK), jnp.int32),
        pltpu.VMEM((NSLOT, CHUNK, EMBED_DIM), jnp.float32),
        pltpu.SemaphoreType.DMA,
        [pltpu.SemaphoreType.DMA] * NSLOT,
        [pltpu.SemaphoreType.DMA] * NSLOT,
    ],
    compiler_params=pltpu.CompilerParams(use_tc_tiling_on_sc=False),
)
def _emb_lookup(idx_hbm, table_hbm, out_hbm, idx_v, rows_v, isem, gsems, wsems):
    wid = lax.axis_index("s") * NUM_CORES + lax.axis_index("c")
    base = wid * PER_W

    pltpu.async_copy(idx_hbm.at[pl.ds(wid * NCHUNK, NCHUNK)], idx_v, isem).wait()

    for b in range(LOOKAHEAD):
        pltpu.async_copy(table_hbm.at[idx_v.at[b]], rows_v.at[b], gsems[b])

    @pl.loop(0, NCHUNK, step=NSLOT)
    def _group(g):
        for b in range(NSLOT):
            j = g + b
            jn = j + LOOKAHEAD
            bn = (b + LOOKAHEAD) % NSLOT

            @pl.when(jn < NCHUNK)
            def _():
                @pl.when(jn >= NSLOT)
                def _():
                    pltpu.make_async_copy(
                        rows_v.at[bn],
                        out_hbm.at[pl.ds(base, CHUNK)],
                        wsems[bn],
                    ).wait()

                pltpu.async_copy(table_hbm.at[idx_v.at[jn]], rows_v.at[bn], gsems[bn])

            pltpu.make_async_copy(
                table_hbm.at[idx_v.at[b]], rows_v.at[b], gsems[b]
            ).wait()
            pltpu.async_copy(
                rows_v.at[b],
                out_hbm.at[pl.ds(base + j * CHUNK, CHUNK)],
                wsems[b],
            )

    for b in range(NSLOT):
        pltpu.make_async_copy(
            rows_v.at[b], out_hbm.at[pl.ds(base, CHUNK)], wsems[b]
        ).wait()


def kernel(x, weight):
    idx2d = x.astype(jnp.int32).reshape(TOTAL // CHUNK, CHUNK)
    wt = weight.T
    tail = jnp.pad(wt[:, NFULL * 128:], ((0, 0), (0, 128 - TAIL)))
    table = _table_transpose(wt, tail).reshape(DICT_SIZE, EMBED_DIM)
    out = _emb_lookup(idx2d, table)
    return out.reshape(BATCH, HIST, EMBED_DIM)


# speedup vs baseline: 1.3108x; 1.3108x over previous
import functools

import jax
import jax.numpy as jnp
from jax import lax
from jax.experimental import pallas as pl
from jax.experimental.pallas import tpu as pltpu
from jax.experimental.pallas import tpu_sc as plsc

DICT_SIZE = 1000000
EMBED_DIM = 64
BATCH = 16384
HIST = 20
TOTAL = BATCH * HIST

NUM_CORES = 2
NUM_SUBCORES = 16
NW = NUM_CORES * NUM_SUBCORES

PER_W = TOTAL // NW
CHUNK = 128
NCHUNK = PER_W // CHUNK
NSLOT = 10
LOOKAHEAD = 5

NFULL = DICT_SIZE // 128
TAIL = DICT_SIZE - NFULL * 128
BLK_FLOOR = NFULL // NW
BLK_EXTRA = NFULL % NW
TSLOT = 4

_mesh = plsc.VectorSubcoreMesh(core_axis_name="c", subcore_axis_name="s")


@functools.partial(
    pl.kernel,
    out_type=jax.ShapeDtypeStruct((DICT_SIZE // 2, 128), jnp.float32),
    mesh=_mesh,
    scratch_types=[
        pltpu.VMEM((TSLOT, 64, 128), jnp.float32),
        pltpu.VMEM((TSLOT, 64, 128), jnp.float32),
        [pltpu.SemaphoreType.DMA] * TSLOT,
        [pltpu.SemaphoreType.DMA] * TSLOT,
    ],
    compiler_params=pltpu.CompilerParams(
        use_tc_tiling_on_sc=True, needs_layout_passes=False
    ),
)
def _table_transpose(wt_hbm, tail_hbm, out_hbm, src_v, dst_v, lsems, ssems):
    wid = lax.axis_index("s") * NUM_CORES + lax.axis_index("c")
    blk0 = wid * BLK_FLOOR + jnp.minimum(wid, BLK_EXTRA)
    nb = BLK_FLOOR + jnp.where(wid < BLK_EXTRA, 1, 0)

    def load(slot, blk):
        pltpu.async_copy(
            wt_hbm.at[:, pl.ds(blk * 128, 128)], src_v.at[slot], lsems[slot]
        )

    def wait_load(slot):
        pltpu.make_async_copy(
            wt_hbm.at[:, pl.ds(0, 128)], src_v.at[slot], lsems[slot]
        ).wait()

    def store(slot, row0, nrows):
        pltpu.async_copy(
            dst_v.at[slot, pl.ds(0, nrows)],
            out_hbm.at[pl.ds(row0, nrows)],
            ssems[slot],
        )

    def wait_store(slot, nrows):
        pltpu.make_async_copy(
            dst_v.at[slot, pl.ds(0, nrows)],
            out_hbm.at[pl.ds(0, nrows)],
            ssems[slot],
        ).wait()

    rowvs = []
    colvs = []
    for j in range(8):
        lane = jax.lax.iota(jnp.int32, 16) + (16 * j)
        rowvs.append(lane // 2)
        colvs.append((lane % 2) * 64)

    def shuffle(slot, ngroups):
        @plsc.parallel_loop(0, 64, step=1, unroll=8)
        def _(d):
            for j in range(ngroups):
                v = src_v[slot, d, pl.ds(16 * j, 16)]
                plsc.store_scatter(dst_v.at[slot], [rowvs[j], colvs[j] + d], v)

    for s in range(TSLOT):
        load(s, blk0 + s)

    @pl.loop(0, BLK_FLOOR + 1, step=TSLOT)
    def _group(g):
        for s in range(TSLOT):
            i = g + s

            @pl.when(i < nb)
            def _():
                wait_load(s)

                @pl.when(i >= TSLOT)
                def _():
                    wait_store(s, 64)

                shuffle(s, 8)
                store(s, (blk0 + i) * 64, 64)

                @pl.when(i + TSLOT < nb)
                def _():
                    load(s, blk0 + i + TSLOT)

    for s in range(TSLOT):
        wait_store(s, 64)

    @pl.when(wid == 0)
    def _():
        pltpu.async_copy(tail_hbm, src_v.at[0], lsems[0])
        pltpu.make_async_copy(tail_hbm, src_v.at[0], lsems[0]).wait()
        shuffle(0, 4)
        store(0, NFULL * 64, TAIL // 2)
        wait_store(0, TAIL // 2)


@functools.partial(
    pl.kernel,
    out_type=jax.ShapeDtypeStruct((TOTAL, EMBED_DIM), jnp.float32),
    mesh=_mesh,
    scratch_types=[
        pltpu.VMEM((NCHUNK, CHUNK), jnp.int32),
        pltpu.VMEM((NSLOT, CHUNK, EMBED_DIM), jnp.float32),
        pltpu.SemaphoreType.DMA,
        [pltpu.SemaphoreType.DMA] * NSLOT,
        [pltpu.SemaphoreType.DMA] * NSLOT,
    ],
    compiler_params=pltpu.CompilerParams(use_tc_tiling_on_sc=False),
)
def _emb_lookup(idx_hbm, table_hbm, out_hbm, idx_v, rows_v, isem, gsems, wsems):
    wid = lax.axis_index("s") * NUM_CORES + lax.axis_index("c")
    base = wid * PER_W

    pltpu.async_copy(idx_hbm.at[pl.ds(wid * NCHUNK, NCHUNK)], idx_v, isem).wait()

    for b in range(LOOKAHEAD):
        pltpu.async_copy(table_hbm.at[idx_v.at[b]], rows_v.at[b], gsems[b])

    @pl.loop(0, NCHUNK, step=NSLOT)
    def _group(g):
        for b in range(NSLOT):
            j = g + b
            jn = j + LOOKAHEAD
            bn = (b + LOOKAHEAD) % NSLOT

            @pl.when(jn < NCHUNK)
            def _():
                @pl.when(jn >= NSLOT)
                def _():
                    pltpu.make_async_copy(
                        rows_v.at[bn],
                        out_hbm.at[pl.ds(base, CHUNK)],
                        wsems[bn],
                    ).wait()

                pltpu.async_copy(table_hbm.at[idx_v.at[jn]], rows_v.at[bn], gsems[bn])

            pltpu.make_async_copy(
                table_hbm.at[idx_v.at[b]], rows_v.at[b], gsems[b]
            ).wait()
            pltpu.async_copy(
                rows_v.at[b],
                out_hbm.at[pl.ds(base + j * CHUNK, CHUNK)],
                wsems[b],
            )

    for b in range(NSLOT):
        pltpu.make_async_copy(
            rows_v.at[b], out_hbm.at[pl.ds(base, CHUNK)], wsems[b]
        ).wait()


def kernel(x, weight):
    idx2d = x.astype(jnp.int32).reshape(TOTAL // CHUNK, CHUNK)
    wt = weight.T
    tail = jnp.pad(wt[:, NFULL * 128:], ((0, 0), (0, 128 - TAIL)))
    table = _table_transpose(wt, tail).reshape(DICT_SIZE, EMBED_DIM)
    out = _emb_lookup(idx2d, table)
    return out.reshape(BATCH, HIST, EMBED_DIM)


# speedup vs baseline: 1.3893x; 1.0599x over previous
import functools

import jax
import jax.numpy as jnp
from jax import lax
from jax.experimental import pallas as pl
from jax.experimental.pallas import tpu as pltpu
from jax.experimental.pallas import tpu_sc as plsc

DICT_SIZE = 1000000
EMBED_DIM = 64
BATCH = 16384
HIST = 20
TOTAL = BATCH * HIST

NUM_CORES = 2
NUM_SUBCORES = 16
NW = NUM_CORES * NUM_SUBCORES

PER_W = TOTAL // NW
CHUNK = 128
NCHUNK = PER_W // CHUNK
NSLOT = 10
LOOKAHEAD = 5

NFULL = DICT_SIZE // 128
TAIL = DICT_SIZE - NFULL * 128
BLK_FLOOR = NFULL // NW
BLK_EXTRA = NFULL % NW
TSLOT = 4

_mesh = plsc.VectorSubcoreMesh(core_axis_name="c", subcore_axis_name="s")


@functools.partial(
    pl.kernel,
    out_type=jax.ShapeDtypeStruct((DICT_SIZE // 2, 128), jnp.float32),
    mesh=_mesh,
    scratch_types=[
        pltpu.VMEM((TSLOT, 64, 131), jnp.float32),
        pltpu.VMEM((TSLOT, 64, 128), jnp.float32),
        [pltpu.SemaphoreType.DMA] * TSLOT,
        [pltpu.SemaphoreType.DMA] * TSLOT,
    ],
    compiler_params=pltpu.CompilerParams(
        use_tc_tiling_on_sc=True, needs_layout_passes=False
    ),
)
def _table_transpose(wt_hbm, tail_hbm, out_hbm, src_v, dst_v, lsems, ssems):
    wid = lax.axis_index("s") * NUM_CORES + lax.axis_index("c")
    blk0 = wid * BLK_FLOOR + jnp.minimum(wid, BLK_EXTRA)
    nb = BLK_FLOOR + jnp.where(wid < BLK_EXTRA, 1, 0)

    def load(slot, blk):
        pltpu.async_copy(
            wt_hbm.at[:, pl.ds(blk * 128, 128)],
            src_v.at[slot, :, pl.ds(0, 128)],
            lsems[slot],
        )

    def wait_load(slot):
        pltpu.make_async_copy(
            wt_hbm.at[:, pl.ds(0, 128)],
            src_v.at[slot, :, pl.ds(0, 128)],
            lsems[slot],
        ).wait()

    def store(slot, row0, nrows):
        pltpu.async_copy(
            dst_v.at[slot, pl.ds(0, nrows)],
            out_hbm.at[pl.ds(row0, nrows)],
            ssems[slot],
        )

    def wait_store(slot, nrows):
        pltpu.make_async_copy(
            dst_v.at[slot, pl.ds(0, nrows)],
            out_hbm.at[pl.ds(0, nrows)],
            ssems[slot],
        ).wait()

    rowvs = [jax.lax.iota(jnp.int32, 16) + 16 * cg for cg in range(4)]
    zero16 = jnp.zeros((16,), jnp.int32)

    def shuffle(slot):
        @plsc.parallel_loop(0, 64, step=1, unroll=8)
        def _(r):
            for p in range(2):
                colv = zero16 + (2 * r + p)
                for cg in range(4):
                    v = plsc.load_gather(src_v.at[slot], [rowvs[cg], colv])
                    dst_v[slot, r, pl.ds(64 * p + 16 * cg, 16)] = v

    for s in range(TSLOT):
        load(s, blk0 + s)

    @pl.loop(0, BLK_FLOOR + 1, step=TSLOT)
    def _group(g):
        for s in range(TSLOT):
            i = g + s

            @pl.when(i < nb)
            def _():
                wait_load(s)

                @pl.when(i >= TSLOT)
                def _():
                    wait_store(s, 64)

                shuffle(s)
                store(s, (blk0 + i) * 64, 64)

                @pl.when(i + TSLOT < nb)
                def _():
                    load(s, blk0 + i + TSLOT)

    for s in range(TSLOT):
        wait_store(s, 64)

    @pl.when(wid == 0)
    def _():
        pltpu.async_copy(tail_hbm, src_v.at[0, :, pl.ds(0, 128)], lsems[0])
        pltpu.make_async_copy(
            tail_hbm, src_v.at[0, :, pl.ds(0, 128)], lsems[0]
        ).wait()
        shuffle(0)
        store(0, NFULL * 64, TAIL // 2)
        wait_store(0, TAIL // 2)


@functools.partial(
    pl.kernel,
    out_type=jax.ShapeDtypeStruct((TOTAL, EMBED_DIM), jnp.float32),
    mesh=_mesh,
    scratch_types=[
        pltpu.VMEM((NCHUNK, CHUNK), jnp.int32),
        pltpu.VMEM((NSLOT, CHUNK, EMBED_DIM), jnp.float32),
        pltpu.SemaphoreType.DMA,
        [pltpu.SemaphoreType.DMA] * NSLOT,
        [pltpu.SemaphoreType.DMA] * NSLOT,
    ],
    compiler_params=pltpu.CompilerParams(use_tc_tiling_on_sc=False),
)
def _emb_lookup(idx_hbm, table_hbm, out_hbm, idx_v, rows_v, isem, gsems, wsems):
    wid = lax.axis_index("s") * NUM_CORES + lax.axis_index("c")
    base = wid * PER_W

    pltpu.async_copy(idx_hbm.at[pl.ds(wid * NCHUNK, NCHUNK)], idx_v, isem).wait()

    for b in range(LOOKAHEAD):
        pltpu.async_copy(table_hbm.at[idx_v.at[b]], rows_v.at[b], gsems[b])

    @pl.loop(0, NCHUNK, step=NSLOT)
    def _group(g):
        for b in range(NSLOT):
            j = g + b
            jn = j + LOOKAHEAD
            bn = (b + LOOKAHEAD) % NSLOT

            @pl.when(jn < NCHUNK)
            def _():
                @pl.when(jn >= NSLOT)
                def _():
                    pltpu.make_async_copy(
                        rows_v.at[bn],
                        out_hbm.at[pl.ds(base, CHUNK)],
                        wsems[bn],
                    ).wait()

                pltpu.async_copy(table_hbm.at[idx_v.at[jn]], rows_v.at[bn], gsems[bn])

            pltpu.make_async_copy(
                table_hbm.at[idx_v.at[b]], rows_v.at[b], gsems[b]
            ).wait()
            pltpu.async_copy(
                rows_v.at[b],
                out_hbm.at[pl.ds(base + j * CHUNK, CHUNK)],
                wsems[b],
            )

    for b in range(NSLOT):
        pltpu.make_async_copy(
            rows_v.at[b], out_hbm.at[pl.ds(base, CHUNK)], wsems[b]
        ).wait()


def kernel(x, weight):
    idx2d = x.astype(jnp.int32).reshape(TOTAL // CHUNK, CHUNK)
    wt = weight.T
    tail = jnp.pad(wt[:, NFULL * 128:], ((0, 0), (0, 128 - TAIL)))
    table = _table_transpose(wt, tail).reshape(DICT_SIZE, EMBED_DIM)
    out = _emb_lookup(idx2d, table)
    return out.reshape(BATCH, HIST, EMBED_DIM)
